# Initial kernel scaffold; baseline (speedup 1.0000x reference)
import functools

import jax
import jax.numpy as jnp
from jax import lax
from jax.experimental import pallas as pl
from jax.experimental.pallas import tpu as pltpu
from jax.experimental.pallas import tpu_sc as plsc

N = 50000
E = 800000
EH = 1600000
D = 64

_NC = 2
_NS = 16
_NW = _NC * _NS
_L = 16

PKW = E // _L
A_CHUNK_E = 6400
A_CHUNK_W = A_CHUNK_E // _L
A_NCHUNKS = E // A_CHUNK_E
A_ITERS = -(-A_NCHUNKS // _NW)


def _stage_a_kernel(nc_hbm, gsrc_hbm, gdst_hbm, pk_hbm,
                    nc_v, src_v, dst_v, pk_v):
    wid = lax.axis_index("s") * _NC + lax.axis_index("c")
    pltpu.sync_copy(nc_hbm, nc_v)
    lanes = lax.iota(jnp.int32, _L)
    shifts = lanes * 2

    def chunk_body(i, _):
        cid = wid + i * _NW

        @pl.when(cid < A_NCHUNKS)
        def _():
            e0 = cid * A_CHUNK_E
            pltpu.sync_copy(gsrc_hbm.at[pl.ds(e0, A_CHUNK_E)], src_v)
            pltpu.sync_copy(gdst_hbm.at[pl.ds(e0, A_CHUNK_E)], dst_v)

            def word_body(j, _):
                sv = src_v[pl.ds(j * _L, _L)]
                dv = dst_v[pl.ds(j * _L, _L)]
                cs = plsc.load_gather(nc_v, [sv])
                cd = plsc.load_gather(nc_v, [dv])
                code = (cs << 1) | cd
                word = jnp.sum(code << shifts)
                pk_v[j] = word
                return 0

            lax.fori_loop(0, A_CHUNK_W, word_body, 0, unroll=4)
            pltpu.sync_copy(pk_v, pk_hbm.at[pl.ds(cid * A_CHUNK_W, A_CHUNK_W)])
        return 0

    lax.fori_loop(0, A_ITERS, chunk_body, 0)


B_CHUNK = 8000
B_NCHUNKS = EH // B_CHUNK
B_ITERS = -(-B_NCHUNKS // _NW)


def _stage_b_kernel(pk_hbm, hsrc_hbm, hdst_hbm, idx_hbm,
                    pk_v, sv_v, dv_v, ix_v):
    wid = lax.axis_index("s") * _NC + lax.axis_index("c")
    pltpu.sync_copy(pk_hbm, pk_v)

    def chunk_body(i, _):
        cid = wid + i * _NW

        @pl.when(cid < B_NCHUNKS)
        def _():
            e0 = cid * B_CHUNK
            pltpu.sync_copy(hsrc_hbm.at[pl.ds(e0, B_CHUNK)], sv_v)
            pltpu.sync_copy(hdst_hbm.at[pl.ds(e0, B_CHUNK)], dv_v)

            def vec_body(j, _):
                sv = sv_v[pl.ds(j * _L, _L)]
                dv = dv_v[pl.ds(j * _L, _L)]
                ws = plsc.load_gather(pk_v, [lax.shift_right_logical(sv, 4)])
                wd = plsc.load_gather(pk_v, [lax.shift_right_logical(dv, 4)])
                cs = lax.shift_right_logical(ws, (sv & 15) << 1) & 3
                lo = lax.shift_right_logical(wd, (dv & 15) << 1) & 1
                ix_v[pl.ds(j * _L, _L)] = (cs << 1) | lo
                return 0

            lax.fori_loop(0, B_CHUNK // _L, vec_body, 0, unroll=4)
            pltpu.sync_copy(ix_v, idx_hbm.at[pl.ds(e0, B_CHUNK)])
        return 0

    lax.fori_loop(0, B_ITERS, chunk_body, 0)


C_ROWS = 8000
C_GRID = EH // C_ROWS


def _stage_c_kernel(idx_ref, e1_ref, e2_ref, e3_ref, out_ref):
    rows = []
    for k in range(8):
        a, b, c = (k >> 2) & 1, (k >> 1) & 1, k & 1
        rows.append(e1_ref[int(a == c)] + e2_ref[int(a == b)]
                    + e3_ref[int(b == c)])
    t8 = jnp.stack(rows)
    idx = idx_ref[...]
    kiota = lax.broadcasted_iota(jnp.int32, (8, C_ROWS), 0)
    oh = (jnp.broadcast_to(idx, (8, C_ROWS)) == kiota).astype(jnp.float32)
    out_ref[...] = lax.dot_general(
        oh, t8, (((0,), (0,)), ((), ())),
        preferred_element_type=jnp.float32)


def kernel(node_colors, g_edge_index, h_edge_index, e1, e2, e3):
    mesh = plsc.VectorSubcoreMesh(core_axis_name="c", subcore_axis_name="s")

    stage_a = pl.kernel(
        _stage_a_kernel,
        mesh=mesh,
        out_type=jax.ShapeDtypeStruct((PKW,), jnp.int32),
        scratch_types=[
            pltpu.VMEM((N,), jnp.int32),
            pltpu.VMEM((A_CHUNK_E,), jnp.int32),
            pltpu.VMEM((A_CHUNK_E,), jnp.int32),
            pltpu.VMEM((A_CHUNK_W,), jnp.int32),
        ],
    )
    pk = stage_a(node_colors, g_edge_index[0], g_edge_index[1])

    stage_b = pl.kernel(
        _stage_b_kernel,
        mesh=mesh,
        out_type=jax.ShapeDtypeStruct((EH,), jnp.int32),
        scratch_types=[
            pltpu.VMEM((PKW,), jnp.int32),
            pltpu.VMEM((B_CHUNK,), jnp.int32),
            pltpu.VMEM((B_CHUNK,), jnp.int32),
            pltpu.VMEM((B_CHUNK,), jnp.int32),
        ],
    )
    idx = stage_b(pk, h_edge_index[0], h_edge_index[1])

    idx2d = idx.reshape(C_GRID, C_ROWS)
    out = pl.pallas_call(
        _stage_c_kernel,
        grid=(C_GRID,),
        in_specs=[
            pl.BlockSpec((1, C_ROWS), lambda g: (g, 0)),
            pl.BlockSpec((2, D), lambda g: (0, 0)),
            pl.BlockSpec((2, D), lambda g: (0, 0)),
            pl.BlockSpec((2, D), lambda g: (0, 0)),
        ],
        out_specs=pl.BlockSpec((C_ROWS, D), lambda g: (g, 0)),
        out_shape=jax.ShapeDtypeStruct((EH, D), jnp.float32),
    )(idx2d, e1, e2, e3)
    return out


# speedup vs baseline: 66.5571x; 66.5571x over previous
import functools

import jax
import jax.numpy as jnp
from jax import lax
from jax.experimental import pallas as pl
from jax.experimental.pallas import tpu as pltpu
from jax.experimental.pallas import tpu_sc as plsc

N = 50000
E = 800000
EH = 1600000
D = 64

_NC = 2
_NS = 16
_NW = _NC * _NS
_L = 16

PKW = E // _L
A_CHUNK_E = 6400
A_CHUNK_W = A_CHUNK_E // _L
A_NCHUNKS = E // A_CHUNK_E
A_ITERS = -(-A_NCHUNKS // _NW)


def _stage_a_kernel(nc_hbm, gsrc_hbm, gdst_hbm, pk_hbm,
                    nc_v, src_v, dst_v, pk_v):
    wid = lax.axis_index("s") * _NC + lax.axis_index("c")
    pltpu.sync_copy(nc_hbm, nc_v)
    lanes = lax.iota(jnp.int32, _L)

    def chunk_body(i, _):
        cid = wid + i * _NW

        @pl.when(cid < A_NCHUNKS)
        def _():
            e0 = cid * A_CHUNK_E
            pltpu.sync_copy(gsrc_hbm.at[pl.ds(e0, A_CHUNK_E)], src_v)
            pltpu.sync_copy(gdst_hbm.at[pl.ds(e0, A_CHUNK_E)], dst_v)

            def group_body(g, _):
                wbase = g * _L
                word = jnp.zeros((_L,), jnp.int32)
                for l in range(_L):
                    pos = (wbase + lanes) * _L + l
                    sv = plsc.load_gather(src_v, [pos])
                    dv = plsc.load_gather(dst_v, [pos])
                    cs = plsc.load_gather(nc_v, [sv])
                    cd = plsc.load_gather(nc_v, [dv])
                    code = (cs << 1) | cd
                    word = word | (code << (2 * l))
                pk_v[pl.ds(wbase, _L)] = word
                return 0

            lax.fori_loop(0, A_CHUNK_W // _L, group_body, 0)
            pltpu.sync_copy(pk_v, pk_hbm.at[pl.ds(cid * A_CHUNK_W, A_CHUNK_W)])
        return 0

    lax.fori_loop(0, A_ITERS, chunk_body, 0)


B_CHUNK = 8000
B_NCHUNKS = EH // B_CHUNK
B_ITERS = -(-B_NCHUNKS // _NW)


def _stage_b_kernel(pk_hbm, hsrc_hbm, hdst_hbm, idx_hbm,
                    pk_v, sv_v, dv_v, ix_v):
    wid = lax.axis_index("s") * _NC + lax.axis_index("c")
    pltpu.sync_copy(pk_hbm, pk_v)

    def chunk_body(i, _):
        cid = wid + i * _NW

        @pl.when(cid < B_NCHUNKS)
        def _():
            e0 = cid * B_CHUNK
            pltpu.sync_copy(hsrc_hbm.at[pl.ds(e0, B_CHUNK)], sv_v)
            pltpu.sync_copy(hdst_hbm.at[pl.ds(e0, B_CHUNK)], dv_v)

            def vec_body(j, _):
                sv = sv_v[pl.ds(j * _L, _L)]
                dv = dv_v[pl.ds(j * _L, _L)]
                ws = plsc.load_gather(pk_v, [lax.shift_right_logical(sv, 4)])
                wd = plsc.load_gather(pk_v, [lax.shift_right_logical(dv, 4)])
                cs = lax.shift_right_logical(ws, (sv & 15) << 1) & 3
                lo = lax.shift_right_logical(wd, (dv & 15) << 1) & 1
                ix_v[pl.ds(j * _L, _L)] = (cs << 1) | lo
                return 0

            lax.fori_loop(0, B_CHUNK // _L, vec_body, 0, unroll=4)
            pltpu.sync_copy(ix_v, idx_hbm.at[pl.ds(e0, B_CHUNK)])
        return 0

    lax.fori_loop(0, B_ITERS, chunk_body, 0)


C_ROWS = 8000
C_GRID = EH // C_ROWS


def _stage_c_kernel(idx_ref, e1_ref, e2_ref, e3_ref, out_ref):
    rows = []
    for k in range(8):
        a, b, c = (k >> 2) & 1, (k >> 1) & 1, k & 1
        rows.append(e1_ref[int(a == c)] + e2_ref[int(a == b)]
                    + e3_ref[int(b == c)])
    t8 = jnp.stack(rows)
    idx = idx_ref[0]
    kiota = lax.broadcasted_iota(jnp.int32, (8, C_ROWS), 0)
    oh = (jnp.broadcast_to(idx, (8, C_ROWS)) == kiota).astype(jnp.float32)
    out_ref[...] = lax.dot_general(
        oh, t8, (((0,), (0,)), ((), ())),
        preferred_element_type=jnp.float32)


def kernel(node_colors, g_edge_index, h_edge_index, e1, e2, e3):
    mesh = plsc.VectorSubcoreMesh(core_axis_name="c", subcore_axis_name="s")
    sc_params = pltpu.CompilerParams(needs_layout_passes=False)

    stage_a = pl.kernel(
        _stage_a_kernel,
        mesh=mesh,
        compiler_params=sc_params,
        out_type=jax.ShapeDtypeStruct((PKW,), jnp.int32),
        scratch_types=[
            pltpu.VMEM((N,), jnp.int32),
            pltpu.VMEM((A_CHUNK_E,), jnp.int32),
            pltpu.VMEM((A_CHUNK_E,), jnp.int32),
            pltpu.VMEM((A_CHUNK_W,), jnp.int32),
        ],
    )
    pk = stage_a(node_colors, g_edge_index[0], g_edge_index[1])

    stage_b = pl.kernel(
        _stage_b_kernel,
        mesh=mesh,
        compiler_params=sc_params,
        out_type=jax.ShapeDtypeStruct((EH,), jnp.int32),
        scratch_types=[
            pltpu.VMEM((PKW,), jnp.int32),
            pltpu.VMEM((B_CHUNK,), jnp.int32),
            pltpu.VMEM((B_CHUNK,), jnp.int32),
            pltpu.VMEM((B_CHUNK,), jnp.int32),
        ],
    )
    idx = stage_b(pk, h_edge_index[0], h_edge_index[1])

    idx3d = idx.reshape(C_GRID, 1, C_ROWS)
    out = pl.pallas_call(
        _stage_c_kernel,
        grid=(C_GRID,),
        in_specs=[
            pl.BlockSpec((1, 1, C_ROWS), lambda g: (g, 0, 0)),
            pl.BlockSpec((2, D), lambda g: (0, 0)),
            pl.BlockSpec((2, D), lambda g: (0, 0)),
            pl.BlockSpec((2, D), lambda g: (0, 0)),
        ],
        out_specs=pl.BlockSpec((C_ROWS, D), lambda g: (g, 0)),
        out_shape=jax.ShapeDtypeStruct((EH, D), jnp.float32),
    )(idx3d, e1, e2, e3)
    return out


# speedup vs baseline: 205.8276x; 3.0925x over previous
import functools

import jax
import jax.numpy as jnp
from jax import lax
from jax.experimental import pallas as pl
from jax.experimental.pallas import tpu as pltpu
from jax.experimental.pallas import tpu_sc as plsc

N = 50000
E = 800000
EH = 1600000
D = 64

_NC = 2
_NS = 16
_NW = _NC * _NS
_L = 16

PKW = E // _L
A_CHUNK_E = 6400
A_CHUNK_W = A_CHUNK_E // _L
A_NCHUNKS = E // A_CHUNK_E
A_ITERS = -(-A_NCHUNKS // _NW)


def _stage_a_kernel(nc_hbm, gsrc_hbm, gdst_hbm, pk_hbm,
                    nc_v, src_v, dst_v, pk_v):
    wid = lax.axis_index("s") * _NC + lax.axis_index("c")
    pltpu.sync_copy(nc_hbm, nc_v)
    lanes = lax.iota(jnp.int32, _L)

    def chunk_body(i, _):
        cid = wid + i * _NW

        @pl.when(cid < A_NCHUNKS)
        def _():
            e0 = cid * A_CHUNK_E
            pltpu.sync_copy(gsrc_hbm.at[pl.ds(e0, A_CHUNK_E)], src_v)
            pltpu.sync_copy(gdst_hbm.at[pl.ds(e0, A_CHUNK_E)], dst_v)

            def group_body(g, _):
                wbase = g * _L
                word = jnp.zeros((_L,), jnp.int32)
                for l in range(_L):
                    pos = (wbase + lanes) * _L + l
                    sv = plsc.load_gather(src_v, [pos])
                    dv = plsc.load_gather(dst_v, [pos])
                    cs = plsc.load_gather(nc_v, [sv])
                    cd = plsc.load_gather(nc_v, [dv])
                    code = (cs << 1) | cd
                    word = word | (code << (2 * l))
                pk_v[pl.ds(wbase, _L)] = word
                return 0

            lax.fori_loop(0, A_CHUNK_W // _L, group_body, 0)
            pltpu.sync_copy(pk_v, pk_hbm.at[pl.ds(cid * A_CHUNK_W, A_CHUNK_W)])
        return 0

    lax.fori_loop(0, A_ITERS, chunk_body, 0)


B_CHUNK = 8000
B_NCHUNKS = EH // B_CHUNK
B_ITERS = -(-B_NCHUNKS // _NW)


def _stage_b_kernel(pk_hbm, hsrc_hbm, hdst_hbm, idx_hbm,
                    pk_v, sv_v, dv_v, ix_v):
    wid = lax.axis_index("s") * _NC + lax.axis_index("c")
    pltpu.sync_copy(pk_hbm, pk_v)

    def chunk_body(i, _):
        cid = wid + i * _NW

        @pl.when(cid < B_NCHUNKS)
        def _():
            e0 = cid * B_CHUNK
            pltpu.sync_copy(hsrc_hbm.at[pl.ds(e0, B_CHUNK)], sv_v)
            pltpu.sync_copy(hdst_hbm.at[pl.ds(e0, B_CHUNK)], dv_v)

            def vec_body(j, _):
                sv = sv_v[pl.ds(j * _L, _L)]
                dv = dv_v[pl.ds(j * _L, _L)]
                ws = plsc.load_gather(pk_v, [lax.shift_right_logical(sv, 4)])
                wd = plsc.load_gather(pk_v, [lax.shift_right_logical(dv, 4)])
                cs = lax.shift_right_logical(ws, (sv & 15) << 1) & 3
                lo = lax.shift_right_logical(wd, (dv & 15) << 1) & 1
                ix_v[pl.ds(j * _L, _L)] = (cs << 1) | lo
                return 0

            lax.fori_loop(0, B_CHUNK // _L, vec_body, 0, unroll=4)
            pltpu.sync_copy(ix_v, idx_hbm.at[pl.ds(e0, B_CHUNK)])
        return 0

    lax.fori_loop(0, B_ITERS, chunk_body, 0)


C_COLS = 12800
C_GRID = EH // C_COLS


def _stage_c_kernel(idx_ref, e1_ref, e2_ref, e3_ref, out_ref):
    rows = []
    for k in range(8):
        a, b, c = (k >> 2) & 1, (k >> 1) & 1, k & 1
        rows.append(e1_ref[int(a == c)] + e2_ref[int(a == b)]
                    + e3_ref[int(b == c)])
    t8 = jnp.stack(rows)
    g = pl.program_id(0)
    idx = idx_ref[pl.ds(g * C_COLS, C_COLS)].reshape(1, C_COLS)
    kiota = lax.broadcasted_iota(jnp.int32, (8, C_COLS), 0)
    oh = (jnp.broadcast_to(idx, (8, C_COLS)) == kiota).astype(jnp.float32)
    out_ref[...] = lax.dot_general(
        t8, oh, (((0,), (0,)), ((), ())),
        preferred_element_type=jnp.float32)


def kernel(node_colors, g_edge_index, h_edge_index, e1, e2, e3):
    mesh = plsc.VectorSubcoreMesh(core_axis_name="c", subcore_axis_name="s")
    sc_params = pltpu.CompilerParams(needs_layout_passes=False)

    stage_a = pl.kernel(
        _stage_a_kernel,
        mesh=mesh,
        compiler_params=sc_params,
        out_type=jax.ShapeDtypeStruct((PKW,), jnp.int32),
        scratch_types=[
            pltpu.VMEM((N,), jnp.int32),
            pltpu.VMEM((A_CHUNK_E,), jnp.int32),
            pltpu.VMEM((A_CHUNK_E,), jnp.int32),
            pltpu.VMEM((A_CHUNK_W,), jnp.int32),
        ],
    )
    pk = stage_a(node_colors, g_edge_index[0], g_edge_index[1])

    stage_b = pl.kernel(
        _stage_b_kernel,
        mesh=mesh,
        compiler_params=sc_params,
        out_type=jax.ShapeDtypeStruct((EH,), jnp.int32),
        scratch_types=[
            pltpu.VMEM((PKW,), jnp.int32),
            pltpu.VMEM((B_CHUNK,), jnp.int32),
            pltpu.VMEM((B_CHUNK,), jnp.int32),
            pltpu.VMEM((B_CHUNK,), jnp.int32),
        ],
    )
    idx = stage_b(pk, h_edge_index[0], h_edge_index[1])

    out_t = pl.pallas_call(
        _stage_c_kernel,
        grid=(C_GRID,),
        in_specs=[
            pl.BlockSpec((EH,), lambda g: (0,)),
            pl.BlockSpec((2, D), lambda g: (0, 0)),
            pl.BlockSpec((2, D), lambda g: (0, 0)),
            pl.BlockSpec((2, D), lambda g: (0, 0)),
        ],
        out_specs=pl.BlockSpec((D, C_COLS), lambda g: (0, g)),
        out_shape=jax.ShapeDtypeStruct((D, EH), jnp.float32),
    )(idx, e1, e2, e3)
    return out_t.T


# speedup vs baseline: 301.3790x; 1.4642x over previous
import functools

import jax
import jax.numpy as jnp
from jax import lax
from jax.experimental import pallas as pl
from jax.experimental.pallas import tpu as pltpu
from jax.experimental.pallas import tpu_sc as plsc

N = 50000
E = 800000
EH = 1600000
D = 64

_NC = 2
_NS = 16
_NW = _NC * _NS
_L = 16

PKW = E // _L
A_CHUNK_E = 6400
A_CHUNK_W = A_CHUNK_E // _L
A_NCHUNKS = E // A_CHUNK_E
A_ITERS = -(-A_NCHUNKS // _NW)


def _stage_a_kernel(nc_hbm, ge_hbm, pk_hbm,
                    nc_v, ge_v, pk_v):
    wid = lax.axis_index("s") * _NC + lax.axis_index("c")
    pltpu.sync_copy(nc_hbm, nc_v)
    lanes = lax.iota(jnp.int32, _L)

    def chunk_body(i, _):
        cid = wid + i * _NW

        @pl.when(cid < A_NCHUNKS)
        def _():
            e0 = cid * A_CHUNK_E
            pltpu.sync_copy(ge_hbm.at[:, pl.ds(e0, A_CHUNK_E)], ge_v)

            def group_body(g, _):
                wbase = g * _L
                word = jnp.zeros((_L,), jnp.int32)
                for l in range(_L):
                    pos = (wbase + lanes) * _L + l
                    sv = plsc.load_gather(ge_v, [jnp.zeros((_L,), jnp.int32),
                                                 pos])
                    dv = plsc.load_gather(ge_v, [jnp.ones((_L,), jnp.int32),
                                                 pos])
                    cs = plsc.load_gather(nc_v, [sv])
                    cd = plsc.load_gather(nc_v, [dv])
                    code = (cs << 1) | cd
                    word = word | (code << (2 * l))
                pk_v[pl.ds(wbase, _L)] = word
                return 0

            lax.fori_loop(0, A_CHUNK_W // _L, group_body, 0)
            pltpu.sync_copy(pk_v, pk_hbm.at[pl.ds(cid * A_CHUNK_W, A_CHUNK_W)])
        return 0

    lax.fori_loop(0, A_ITERS, chunk_body, 0)


B_CHUNK = 12800
B_NCHUNKS = EH // B_CHUNK
B_ITERS = -(-B_NCHUNKS // _NW)


def _stage_b_kernel(pk_hbm, he_hbm, idx_hbm,
                    pk_v, he_v, ix_v):
    wid = lax.axis_index("s") * _NC + lax.axis_index("c")
    pltpu.sync_copy(pk_hbm, pk_v)

    def chunk_body(i, _):
        cid = wid + i * _NW

        @pl.when(cid < B_NCHUNKS)
        def _():
            e0 = cid * B_CHUNK
            pltpu.sync_copy(he_hbm.at[:, pl.ds(e0, B_CHUNK)], he_v)

            def vec_body(j, _):
                sv = he_v[0, pl.ds(j * _L, _L)]
                dv = he_v[1, pl.ds(j * _L, _L)]
                ws = plsc.load_gather(pk_v, [lax.shift_right_logical(sv, 4)])
                wd = plsc.load_gather(pk_v, [lax.shift_right_logical(dv, 4)])
                cs = lax.shift_right_logical(ws, (sv & 15) << 1) & 3
                lo = lax.shift_right_logical(wd, (dv & 15) << 1) & 1
                ix_v[pl.ds(j * _L, _L)] = (cs << 1) | lo
                return 0

            lax.fori_loop(0, B_CHUNK // _L, vec_body, 0, unroll=4)
            pltpu.sync_copy(ix_v, idx_hbm.at[pl.ds(e0, B_CHUNK)])
        return 0

    lax.fori_loop(0, B_ITERS, chunk_body, 0)


C_COLS = 32000
C_GRID = EH // C_COLS


def _stage_c_kernel(idx_ref, e1_ref, e2_ref, e3_ref, out_ref):
    rows = []
    for k in range(8):
        a, b, c = (k >> 2) & 1, (k >> 1) & 1, k & 1
        rows.append(e1_ref[int(a == c)] + e2_ref[int(a == b)]
                    + e3_ref[int(b == c)])
    t8 = jnp.stack(rows)
    g = pl.program_id(0)
    idx = idx_ref[pl.ds(g * C_COLS, C_COLS)].reshape(1, C_COLS)
    kiota = lax.broadcasted_iota(jnp.int32, (8, C_COLS), 0)
    oh = (jnp.broadcast_to(idx, (8, C_COLS)) == kiota).astype(jnp.float32)
    out_ref[...] = lax.dot_general(
        t8, oh, (((0,), (0,)), ((), ())),
        preferred_element_type=jnp.float32)


def kernel(node_colors, g_edge_index, h_edge_index, e1, e2, e3):
    mesh = plsc.VectorSubcoreMesh(core_axis_name="c", subcore_axis_name="s")
    sc_params = pltpu.CompilerParams(needs_layout_passes=False)

    stage_a = pl.kernel(
        _stage_a_kernel,
        mesh=mesh,
        compiler_params=sc_params,
        out_type=jax.ShapeDtypeStruct((PKW,), jnp.int32),
        scratch_types=[
            pltpu.VMEM((N,), jnp.int32),
            pltpu.VMEM((2, A_CHUNK_E), jnp.int32),
            pltpu.VMEM((A_CHUNK_W,), jnp.int32),
        ],
    )
    pk = stage_a(node_colors, g_edge_index)

    stage_b = pl.kernel(
        _stage_b_kernel,
        mesh=mesh,
        compiler_params=sc_params,
        out_type=jax.ShapeDtypeStruct((EH,), jnp.int32),
        scratch_types=[
            pltpu.VMEM((PKW,), jnp.int32),
            pltpu.VMEM((2, B_CHUNK), jnp.int32),
            pltpu.VMEM((B_CHUNK,), jnp.int32),
        ],
    )
    idx = stage_b(pk, h_edge_index)

    out_t = pl.pallas_call(
        _stage_c_kernel,
        grid=(C_GRID,),
        in_specs=[
            pl.BlockSpec((EH,), lambda g: (0,)),
            pl.BlockSpec((2, D), lambda g: (0, 0)),
            pl.BlockSpec((2, D), lambda g: (0, 0)),
            pl.BlockSpec((2, D), lambda g: (0, 0)),
        ],
        out_specs=pl.BlockSpec((D, C_COLS), lambda g: (0, g)),
        out_shape=jax.ShapeDtypeStruct((D, EH), jnp.float32),
    )(idx, e1, e2, e3)
    return out_t.T


# speedup vs baseline: 314.4849x; 1.0435x over previous
import functools

import jax
import jax.numpy as jnp
from jax import lax
from jax.experimental import pallas as pl
from jax.experimental.pallas import tpu as pltpu
from jax.experimental.pallas import tpu_sc as plsc

N = 50000
E = 800000
EH = 1600000
D = 64

_NC = 2
_NS = 16
_NW = _NC * _NS
_L = 16

PKW = E // _L
A_CHUNK_E = 6400
A_CHUNK_W = A_CHUNK_E // _L
A_NCHUNKS = E // A_CHUNK_E
A_ITERS = -(-A_NCHUNKS // _NW)


def _stage_a_kernel(nc_hbm, ge_hbm, pk_hbm,
                    nc_v, ge0, ge1, pk0, pk1, sin0, sin1, sout0, sout1):
    wid = lax.axis_index("s") * _NC + lax.axis_index("c")
    lanes = lax.iota(jnp.int32, _L)
    ges = (ge0, ge1)
    pks = (pk0, pk1)
    sins = (sin0, sin1)
    souts = (sout0, sout1)

    def cid(r):
        return wid + r * _NW

    def in_copy(r):
        b = r % 2
        return pltpu.make_async_copy(
            ge_hbm.at[:, pl.ds(cid(r) * A_CHUNK_E, A_CHUNK_E)],
            ges[b], sins[b])

    def out_copy(r):
        b = r % 2
        return pltpu.make_async_copy(
            pks[b],
            pk_hbm.at[pl.ds(cid(r) * A_CHUNK_W, A_CHUNK_W)], souts[b])

    def start_in(r):
        @pl.when(cid(r) < A_NCHUNKS)
        def _():
            in_copy(r).start()

    start_in(0)
    pltpu.sync_copy(nc_hbm, nc_v)

    for r in range(A_ITERS):
        b = r % 2
        if r + 1 < A_ITERS:
            start_in(r + 1)

        @pl.when(cid(r) < A_NCHUNKS)
        def _(r=r, b=b):
            in_copy(r).wait()
            if r >= 2:
                out_copy(r - 2).wait()
            ge_v = ges[b]
            pk_v = pks[b]

            def group_body(g, _):
                wbase = g * _L
                word = jnp.zeros((_L,), jnp.int32)
                for l in range(_L):
                    pos = (wbase + lanes) * _L + l
                    sv = plsc.load_gather(
                        ge_v, [jnp.zeros((_L,), jnp.int32), pos])
                    dv = plsc.load_gather(
                        ge_v, [jnp.ones((_L,), jnp.int32), pos])
                    cs = plsc.load_gather(nc_v, [sv])
                    cd = plsc.load_gather(nc_v, [dv])
                    code = (cs << 1) | cd
                    word = word | (code << (2 * l))
                pk_v[pl.ds(wbase, _L)] = word
                return 0

            lax.fori_loop(0, A_CHUNK_W // _L, group_body, 0)
            out_copy(r).start()

    for r in range(max(A_ITERS - 2, 0), A_ITERS):
        @pl.when(cid(r) < A_NCHUNKS)
        def _(r=r):
            out_copy(r).wait()


B_CHUNK = 6400
B_NCHUNKS = EH // B_CHUNK
B_ITERS = -(-B_NCHUNKS // _NW)


def _stage_b_kernel(pk_hbm, he_hbm, idx_hbm,
                    pk_v, he0, he1, ix0, ix1, sin0, sin1, sout0, sout1):
    wid = lax.axis_index("s") * _NC + lax.axis_index("c")
    hes = (he0, he1)
    ixs = (ix0, ix1)
    sins = (sin0, sin1)
    souts = (sout0, sout1)

    def cid(r):
        return wid + r * _NW

    def in_copy(r):
        b = r % 2
        return pltpu.make_async_copy(
            he_hbm.at[:, pl.ds(cid(r) * B_CHUNK, B_CHUNK)], hes[b], sins[b])

    def out_copy(r):
        b = r % 2
        return pltpu.make_async_copy(
            ixs[b], idx_hbm.at[pl.ds(cid(r) * B_CHUNK, B_CHUNK)], souts[b])

    def start_in(r):
        @pl.when(cid(r) < B_NCHUNKS)
        def _():
            in_copy(r).start()

    start_in(0)
    pltpu.sync_copy(pk_hbm, pk_v)

    for r in range(B_ITERS):
        b = r % 2
        if r + 1 < B_ITERS:
            start_in(r + 1)

        @pl.when(cid(r) < B_NCHUNKS)
        def _(r=r, b=b):
            in_copy(r).wait()
            if r >= 2:
                out_copy(r - 2).wait()
            he_v = hes[b]
            ix_v = ixs[b]

            def vec_body(j, _):
                sv = he_v[0, pl.ds(j * _L, _L)]
                dv = he_v[1, pl.ds(j * _L, _L)]
                ws = plsc.load_gather(pk_v, [lax.shift_right_logical(sv, 4)])
                wd = plsc.load_gather(pk_v, [lax.shift_right_logical(dv, 4)])
                cs = lax.shift_right_logical(ws, (sv & 15) << 1) & 3
                lo = lax.shift_right_logical(wd, (dv & 15) << 1) & 1
                ix_v[pl.ds(j * _L, _L)] = (cs << 1) | lo
                return 0

            lax.fori_loop(0, B_CHUNK // _L, vec_body, 0, unroll=4)
            out_copy(r).start()

    for r in range(max(B_ITERS - 2, 0), B_ITERS):
        @pl.when(cid(r) < B_NCHUNKS)
        def _(r=r):
            out_copy(r).wait()


C_COLS = 32000
C_GRID = EH // C_COLS


def _stage_c_kernel(idx_ref, e1_ref, e2_ref, e3_ref, out_ref):
    rows = []
    for k in range(8):
        a, b, c = (k >> 2) & 1, (k >> 1) & 1, k & 1
        rows.append(e1_ref[int(a == c)] + e2_ref[int(a == b)]
                    + e3_ref[int(b == c)])
    t8 = jnp.stack(rows)
    g = pl.program_id(0)
    idx = idx_ref[pl.ds(g * C_COLS, C_COLS)].reshape(1, C_COLS)
    kiota = lax.broadcasted_iota(jnp.int32, (8, C_COLS), 0)
    oh = (jnp.broadcast_to(idx, (8, C_COLS)) == kiota).astype(jnp.float32)
    out_ref[...] = lax.dot_general(
        t8, oh, (((0,), (0,)), ((), ())),
        preferred_element_type=jnp.float32)


def kernel(node_colors, g_edge_index, h_edge_index, e1, e2, e3):
    mesh = plsc.VectorSubcoreMesh(core_axis_name="c", subcore_axis_name="s")
    sc_params = pltpu.CompilerParams(needs_layout_passes=False)

    stage_a = pl.kernel(
        _stage_a_kernel,
        mesh=mesh,
        compiler_params=sc_params,
        out_type=jax.ShapeDtypeStruct((PKW,), jnp.int32),
        scratch_types=[
            pltpu.VMEM((N,), jnp.int32),
            pltpu.VMEM((2, A_CHUNK_E), jnp.int32),
            pltpu.VMEM((2, A_CHUNK_E), jnp.int32),
            pltpu.VMEM((A_CHUNK_W,), jnp.int32),
            pltpu.VMEM((A_CHUNK_W,), jnp.int32),
            pltpu.SemaphoreType.DMA,
            pltpu.SemaphoreType.DMA,
            pltpu.SemaphoreType.DMA,
            pltpu.SemaphoreType.DMA,
        ],
    )
    pk = stage_a(node_colors, g_edge_index)

    stage_b = pl.kernel(
        _stage_b_kernel,
        mesh=mesh,
        compiler_params=sc_params,
        out_type=jax.ShapeDtypeStruct((EH,), jnp.int32),
        scratch_types=[
            pltpu.VMEM((PKW,), jnp.int32),
            pltpu.VMEM((2, B_CHUNK), jnp.int32),
            pltpu.VMEM((2, B_CHUNK), jnp.int32),
            pltpu.VMEM((B_CHUNK,), jnp.int32),
            pltpu.VMEM((B_CHUNK,), jnp.int32),
            pltpu.SemaphoreType.DMA,
            pltpu.SemaphoreType.DMA,
            pltpu.SemaphoreType.DMA,
            pltpu.SemaphoreType.DMA,
        ],
    )
    idx = stage_b(pk, h_edge_index)

    out_t = pl.pallas_call(
        _stage_c_kernel,
        grid=(C_GRID,),
        in_specs=[
            pl.BlockSpec((EH,), lambda g: (0,)),
            pl.BlockSpec((2, D), lambda g: (0, 0)),
            pl.BlockSpec((2, D), lambda g: (0, 0)),
            pl.BlockSpec((2, D), lambda g: (0, 0)),
        ],
        out_specs=pl.BlockSpec((D, C_COLS), lambda g: (0, g)),
        out_shape=jax.ShapeDtypeStruct((D, EH), jnp.float32),
    )(idx, e1, e2, e3)
    return out_t.T


# speedup vs baseline: 354.2857x; 1.1266x over previous
import functools

import jax
import jax.numpy as jnp
from jax import lax
from jax.experimental import pallas as pl
from jax.experimental.pallas import tpu as pltpu
from jax.experimental.pallas import tpu_sc as plsc

N = 50000
E = 800000
EH = 1600000
D = 64

_NC = 2
_NS = 16
_NW = _NC * _NS
_L = 16

PKW = E // _L
A_CHUNK_E = 6400
A_CHUNK_W = A_CHUNK_E // _L
A_NCHUNKS = E // A_CHUNK_E
A_ITERS = -(-A_NCHUNKS // _NW)


def _stage_a_kernel(nc_hbm, ge_hbm, pk_hbm,
                    nc_v, ge0, ge1, pk0, pk1, sin0, sin1, sout0, sout1):
    wid = lax.axis_index("s") * _NC + lax.axis_index("c")
    lanes = lax.iota(jnp.int32, _L)
    ges = (ge0, ge1)
    pks = (pk0, pk1)
    sins = (sin0, sin1)
    souts = (sout0, sout1)

    def cid(r):
        return wid + r * _NW

    def in_copy(r):
        b = r % 2
        return pltpu.make_async_copy(
            ge_hbm.at[:, pl.ds(cid(r) * A_CHUNK_E, A_CHUNK_E)],
            ges[b], sins[b])

    def out_copy(r):
        b = r % 2
        return pltpu.make_async_copy(
            pks[b],
            pk_hbm.at[pl.ds(cid(r) * A_CHUNK_W, A_CHUNK_W)], souts[b])

    def start_in(r):
        @pl.when(cid(r) < A_NCHUNKS)
        def _():
            in_copy(r).start()

    start_in(0)
    pltpu.sync_copy(nc_hbm, nc_v)

    for r in range(A_ITERS):
        b = r % 2
        if r + 1 < A_ITERS:
            start_in(r + 1)

        @pl.when(cid(r) < A_NCHUNKS)
        def _(r=r, b=b):
            in_copy(r).wait()
            if r >= 2:
                out_copy(r - 2).wait()
            ge_v = ges[b]
            pk_v = pks[b]

            @plsc.parallel_loop(0, A_CHUNK_W // _L, unroll=2)
            def group_body(g):
                wbase = g * _L
                word = jnp.zeros((_L,), jnp.int32)
                for l in range(_L):
                    pos = (wbase + lanes) * _L + l
                    sv = plsc.load_gather(
                        ge_v, [jnp.zeros((_L,), jnp.int32), pos])
                    dv = plsc.load_gather(
                        ge_v, [jnp.ones((_L,), jnp.int32), pos])
                    cs = plsc.load_gather(nc_v, [sv])
                    cd = plsc.load_gather(nc_v, [dv])
                    code = (cs << 1) | cd
                    word = word | (code << (2 * l))
                pk_v[pl.ds(wbase, _L)] = word

            out_copy(r).start()

    for r in range(max(A_ITERS - 2, 0), A_ITERS):
        @pl.when(cid(r) < A_NCHUNKS)
        def _(r=r):
            out_copy(r).wait()


B_CHUNK = 6400
B_NCHUNKS = EH // B_CHUNK
B_ITERS = -(-B_NCHUNKS // _NW)


def _stage_b_kernel(pk_hbm, he_hbm, idx_hbm,
                    pk_v, he0, he1, ix0, ix1, sin0, sin1, sout0, sout1):
    wid = lax.axis_index("s") * _NC + lax.axis_index("c")
    hes = (he0, he1)
    ixs = (ix0, ix1)
    sins = (sin0, sin1)
    souts = (sout0, sout1)

    def cid(r):
        return wid + r * _NW

    def in_copy(r):
        b = r % 2
        return pltpu.make_async_copy(
            he_hbm.at[:, pl.ds(cid(r) * B_CHUNK, B_CHUNK)], hes[b], sins[b])

    def out_copy(r):
        b = r % 2
        return pltpu.make_async_copy(
            ixs[b], idx_hbm.at[pl.ds(cid(r) * B_CHUNK, B_CHUNK)], souts[b])

    def start_in(r):
        @pl.when(cid(r) < B_NCHUNKS)
        def _():
            in_copy(r).start()

    start_in(0)
    pltpu.sync_copy(pk_hbm, pk_v)

    for r in range(B_ITERS):
        b = r % 2
        if r + 1 < B_ITERS:
            start_in(r + 1)

        @pl.when(cid(r) < B_NCHUNKS)
        def _(r=r, b=b):
            in_copy(r).wait()
            if r >= 2:
                out_copy(r - 2).wait()
            he_v = hes[b]
            ix_v = ixs[b]

            @plsc.parallel_loop(0, B_CHUNK // _L, unroll=8)
            def vec_body(j):
                sv = he_v[0, pl.ds(j * _L, _L)]
                dv = he_v[1, pl.ds(j * _L, _L)]
                ws = plsc.load_gather(pk_v, [lax.shift_right_logical(sv, 4)])
                wd = plsc.load_gather(pk_v, [lax.shift_right_logical(dv, 4)])
                cs = lax.shift_right_logical(ws, (sv & 15) << 1) & 3
                lo = lax.shift_right_logical(wd, (dv & 15) << 1) & 1
                ix_v[pl.ds(j * _L, _L)] = (cs << 1) | lo

            out_copy(r).start()

    for r in range(max(B_ITERS - 2, 0), B_ITERS):
        @pl.when(cid(r) < B_NCHUNKS)
        def _(r=r):
            out_copy(r).wait()


C_COLS = 32000
C_GRID = EH // C_COLS


def _stage_c_kernel(idx_ref, e1_ref, e2_ref, e3_ref, out_ref):
    rows = []
    for k in range(8):
        a, b, c = (k >> 2) & 1, (k >> 1) & 1, k & 1
        rows.append(e1_ref[int(a == c)] + e2_ref[int(a == b)]
                    + e3_ref[int(b == c)])
    t8 = jnp.stack(rows)
    g = pl.program_id(0)
    idx = idx_ref[pl.ds(g * C_COLS, C_COLS)].reshape(1, C_COLS)
    kiota = lax.broadcasted_iota(jnp.int32, (8, C_COLS), 0)
    oh = (jnp.broadcast_to(idx, (8, C_COLS)) == kiota).astype(jnp.float32)
    out_ref[...] = lax.dot_general(
        t8, oh, (((0,), (0,)), ((), ())),
        preferred_element_type=jnp.float32)


def kernel(node_colors, g_edge_index, h_edge_index, e1, e2, e3):
    mesh = plsc.VectorSubcoreMesh(core_axis_name="c", subcore_axis_name="s")
    sc_params = pltpu.CompilerParams(needs_layout_passes=False)

    stage_a = pl.kernel(
        _stage_a_kernel,
        mesh=mesh,
        compiler_params=sc_params,
        out_type=jax.ShapeDtypeStruct((PKW,), jnp.int32),
        scratch_types=[
            pltpu.VMEM((N,), jnp.int32),
            pltpu.VMEM((2, A_CHUNK_E), jnp.int32),
            pltpu.VMEM((2, A_CHUNK_E), jnp.int32),
            pltpu.VMEM((A_CHUNK_W,), jnp.int32),
            pltpu.VMEM((A_CHUNK_W,), jnp.int32),
            pltpu.SemaphoreType.DMA,
            pltpu.SemaphoreType.DMA,
            pltpu.SemaphoreType.DMA,
            pltpu.SemaphoreType.DMA,
        ],
    )
    pk = stage_a(node_colors, g_edge_index)

    stage_b = pl.kernel(
        _stage_b_kernel,
        mesh=mesh,
        compiler_params=sc_params,
        out_type=jax.ShapeDtypeStruct((EH,), jnp.int32),
        scratch_types=[
            pltpu.VMEM((PKW,), jnp.int32),
            pltpu.VMEM((2, B_CHUNK), jnp.int32),
            pltpu.VMEM((2, B_CHUNK), jnp.int32),
            pltpu.VMEM((B_CHUNK,), jnp.int32),
            pltpu.VMEM((B_CHUNK,), jnp.int32),
            pltpu.SemaphoreType.DMA,
            pltpu.SemaphoreType.DMA,
            pltpu.SemaphoreType.DMA,
            pltpu.SemaphoreType.DMA,
        ],
    )
    idx = stage_b(pk, h_edge_index)

    out_t = pl.pallas_call(
        _stage_c_kernel,
        grid=(C_GRID,),
        in_specs=[
            pl.BlockSpec((EH,), lambda g: (0,)),
            pl.BlockSpec((2, D), lambda g: (0, 0)),
            pl.BlockSpec((2, D), lambda g: (0, 0)),
            pl.BlockSpec((2, D), lambda g: (0, 0)),
        ],
        out_specs=pl.BlockSpec((D, C_COLS), lambda g: (0, g)),
        out_shape=jax.ShapeDtypeStruct((D, EH), jnp.float32),
    )(idx, e1, e2, e3)
    return out_t.T


# speedup vs baseline: 372.1849x; 1.0505x over previous
import functools

import jax
import jax.numpy as jnp
from jax import lax
from jax.experimental import pallas as pl
from jax.experimental.pallas import tpu as pltpu
from jax.experimental.pallas import tpu_sc as plsc

N = 50000
E = 800000
EH = 1600000
D = 64

_NC = 2
_NS = 16
_NW = _NC * _NS
_L = 16

PKW = E // _L
A_CHUNK_E = 6400
A_CHUNK_W = A_CHUNK_E // _L
A_NCHUNKS = E // A_CHUNK_E
A_ITERS = -(-A_NCHUNKS // _NW)


def _stage_a_kernel(nc_hbm, ge_hbm, pk_hbm,
                    nc_v, ge0, ge1, pk0, pk1, sin0, sin1, sout0, sout1):
    wid = lax.axis_index("s") * _NC + lax.axis_index("c")
    lanes = lax.iota(jnp.int32, _L)
    ges = (ge0, ge1)
    pks = (pk0, pk1)
    sins = (sin0, sin1)
    souts = (sout0, sout1)

    def cid(r):
        return wid + r * _NW

    def in_copy(r):
        b = r % 2
        return pltpu.make_async_copy(
            ge_hbm.at[:, pl.ds(cid(r) * A_CHUNK_E, A_CHUNK_E)],
            ges[b], sins[b])

    def out_copy(r):
        b = r % 2
        return pltpu.make_async_copy(
            pks[b],
            pk_hbm.at[pl.ds(cid(r) * A_CHUNK_W, A_CHUNK_W)], souts[b])

    def start_in(r):
        @pl.when(cid(r) < A_NCHUNKS)
        def _():
            in_copy(r).start()

    start_in(0)
    pltpu.sync_copy(nc_hbm, nc_v)

    for r in range(A_ITERS):
        b = r % 2
        if r + 1 < A_ITERS:
            start_in(r + 1)

        @pl.when(cid(r) < A_NCHUNKS)
        def _(r=r, b=b):
            in_copy(r).wait()
            if r >= 2:
                out_copy(r - 2).wait()
            ge_v = ges[b]
            pk_v = pks[b]

            @plsc.parallel_loop(0, A_CHUNK_W // _L, unroll=2)
            def group_body(g):
                base = g * (_L * _L)
                word = jnp.zeros((_L,), jnp.int32)
                for j in range(_L):
                    sv = ge_v[0, pl.ds(base + j * _L, _L)]
                    dv = ge_v[1, pl.ds(base + j * _L, _L)]
                    cs = plsc.load_gather(nc_v, [sv])
                    cd = plsc.load_gather(nc_v, [dv])
                    code = (cs << 1) | cd
                    word = word | (code << (2 * j))
                pk_v[pl.ds(g * _L, _L)] = word

            out_copy(r).start()

    for r in range(max(A_ITERS - 2, 0), A_ITERS):
        @pl.when(cid(r) < A_NCHUNKS)
        def _(r=r):
            out_copy(r).wait()


B_CHUNK = 6400
B_NCHUNKS = EH // B_CHUNK
B_ITERS = -(-B_NCHUNKS // _NW)


def _stage_b_kernel(pk_hbm, he_hbm, idx_hbm,
                    pk_v, he0, he1, ix0, ix1, sin0, sin1, sout0, sout1):
    wid = lax.axis_index("s") * _NC + lax.axis_index("c")
    hes = (he0, he1)
    ixs = (ix0, ix1)
    sins = (sin0, sin1)
    souts = (sout0, sout1)

    def cid(r):
        return wid + r * _NW

    def in_copy(r):
        b = r % 2
        return pltpu.make_async_copy(
            he_hbm.at[:, pl.ds(cid(r) * B_CHUNK, B_CHUNK)], hes[b], sins[b])

    def out_copy(r):
        b = r % 2
        return pltpu.make_async_copy(
            ixs[b], idx_hbm.at[pl.ds(cid(r) * B_CHUNK, B_CHUNK)], souts[b])

    def start_in(r):
        @pl.when(cid(r) < B_NCHUNKS)
        def _():
            in_copy(r).start()

    start_in(0)
    pltpu.sync_copy(pk_hbm, pk_v)

    for r in range(B_ITERS):
        b = r % 2
        if r + 1 < B_ITERS:
            start_in(r + 1)

        @pl.when(cid(r) < B_NCHUNKS)
        def _(r=r, b=b):
            in_copy(r).wait()
            if r >= 2:
                out_copy(r - 2).wait()
            he_v = hes[b]
            ix_v = ixs[b]

            @plsc.parallel_loop(0, B_CHUNK // _L, unroll=8)
            def vec_body(j):
                sv = he_v[0, pl.ds(j * _L, _L)]
                dv = he_v[1, pl.ds(j * _L, _L)]
                wis = (lax.shift_right_logical(sv, 8) << 4) | (sv & 15)
                wid_ = (lax.shift_right_logical(dv, 8) << 4) | (dv & 15)
                ws = plsc.load_gather(pk_v, [wis])
                wd = plsc.load_gather(pk_v, [wid_])
                shs = (lax.shift_right_logical(sv, 4) & 15) << 1
                shd = (lax.shift_right_logical(dv, 4) & 15) << 1
                cs = lax.shift_right_logical(ws, shs) & 3
                lo = lax.shift_right_logical(wd, shd) & 1
                ix_v[pl.ds(j * _L, _L)] = (cs << 1) | lo

            out_copy(r).start()

    for r in range(max(B_ITERS - 2, 0), B_ITERS):
        @pl.when(cid(r) < B_NCHUNKS)
        def _(r=r):
            out_copy(r).wait()


C_COLS = 32000
C_GRID = EH // C_COLS


def _stage_c_kernel(idx_ref, e1_ref, e2_ref, e3_ref, out_ref):
    rows = []
    for k in range(8):
        a, b, c = (k >> 2) & 1, (k >> 1) & 1, k & 1
        rows.append(e1_ref[int(a == c)] + e2_ref[int(a == b)]
                    + e3_ref[int(b == c)])
    t8 = jnp.stack(rows)
    g = pl.program_id(0)
    idx = idx_ref[pl.ds(g * C_COLS, C_COLS)].reshape(1, C_COLS)
    kiota = lax.broadcasted_iota(jnp.int32, (8, C_COLS), 0)
    oh = (jnp.broadcast_to(idx, (8, C_COLS)) == kiota).astype(jnp.float32)
    out_ref[...] = lax.dot_general(
        t8, oh, (((0,), (0,)), ((), ())),
        preferred_element_type=jnp.float32)


def kernel(node_colors, g_edge_index, h_edge_index, e1, e2, e3):
    mesh = plsc.VectorSubcoreMesh(core_axis_name="c", subcore_axis_name="s")
    sc_params = pltpu.CompilerParams(needs_layout_passes=False)

    stage_a = pl.kernel(
        _stage_a_kernel,
        mesh=mesh,
        compiler_params=sc_params,
        out_type=jax.ShapeDtypeStruct((PKW,), jnp.int32),
        scratch_types=[
            pltpu.VMEM((N,), jnp.int32),
            pltpu.VMEM((2, A_CHUNK_E), jnp.int32),
            pltpu.VMEM((2, A_CHUNK_E), jnp.int32),
            pltpu.VMEM((A_CHUNK_W,), jnp.int32),
            pltpu.VMEM((A_CHUNK_W,), jnp.int32),
            pltpu.SemaphoreType.DMA,
            pltpu.SemaphoreType.DMA,
            pltpu.SemaphoreType.DMA,
            pltpu.SemaphoreType.DMA,
        ],
    )
    pk = stage_a(node_colors, g_edge_index)

    stage_b = pl.kernel(
        _stage_b_kernel,
        mesh=mesh,
        compiler_params=sc_params,
        out_type=jax.ShapeDtypeStruct((EH,), jnp.int32),
        scratch_types=[
            pltpu.VMEM((PKW,), jnp.int32),
            pltpu.VMEM((2, B_CHUNK), jnp.int32),
            pltpu.VMEM((2, B_CHUNK), jnp.int32),
            pltpu.VMEM((B_CHUNK,), jnp.int32),
            pltpu.VMEM((B_CHUNK,), jnp.int32),
            pltpu.SemaphoreType.DMA,
            pltpu.SemaphoreType.DMA,
            pltpu.SemaphoreType.DMA,
            pltpu.SemaphoreType.DMA,
        ],
    )
    idx = stage_b(pk, h_edge_index)

    out_t = pl.pallas_call(
        _stage_c_kernel,
        grid=(C_GRID,),
        in_specs=[
            pl.BlockSpec((EH,), lambda g: (0,)),
            pl.BlockSpec((2, D), lambda g: (0, 0)),
            pl.BlockSpec((2, D), lambda g: (0, 0)),
            pl.BlockSpec((2, D), lambda g: (0, 0)),
        ],
        out_specs=pl.BlockSpec((D, C_COLS), lambda g: (0, g)),
        out_shape=jax.ShapeDtypeStruct((D, EH), jnp.float32),
    )(idx, e1, e2, e3)
    return out_t.T
